# baseline (speedup 1.0000x reference)
import functools

import jax
import jax.numpy as jnp
from jax import lax
from jax.experimental import pallas as pl
from jax.experimental.pallas import tpu as pltpu
from jax.experimental.pallas import tpu_sc as plsc

_EMB = 128
_NB_REL = 100
_NB_ENT = 1000
_BATCH = 4096
_NB_NEG = 64

_NC = 2
_NS = 16
_NW = _NC * _NS
_C = 32
_POS_PER_W = _BATCH // _NW
_NEG_ROWS_PER_W = _NB_NEG // _NW
_NEG_CHUNKS = _NEG_ROWS_PER_W * _BATCH // _C


def _tables_body(rhb, rhw, rhs, rtb, rtw, rts, eb, ebp,
                 box_ref, t1_ref, t2_ref):
    def corners(base_ref, width_ref, scale_ref):
        w = width_ref[...]
        denom = jnp.maximum(jnp.sum(jnp.abs(w), axis=-1, keepdims=True), 1e-12)
        s = scale_ref[...]
        elu1 = jnp.where(s > 0, s, jnp.exp(jnp.minimum(s, 0.0)) - 1.0) + 1.0
        delta = jnp.abs((w / denom) * elu1)
        b = base_ref[...]
        return b + delta, b - delta

    hu, hl = corners(rhb, rhw, rhs)
    tu, tl = corners(rtb, rtw, rts)
    box_ref[...] = jnp.stack(
        [jnp.stack([hu, hl], axis=1), jnp.stack([tu, tl], axis=1)], axis=1)
    bases = eb[...]
    bumps = ebp[...]
    t1_ref[...] = jnp.stack([bases, bumps], axis=1)
    t2_ref[...] = jnp.stack([bumps, bases], axis=1)


def _make_tables(rhb, rhw, rhs, rtb, rtw, rts, eb, ebp):
    return pl.pallas_call(
        _tables_body,
        out_shape=(
            jax.ShapeDtypeStruct((_NB_REL, 2, 2, _EMB), jnp.float32),
            jax.ShapeDtypeStruct((_NB_ENT, 2, _EMB), jnp.float32),
            jax.ShapeDtypeStruct((_NB_ENT, 2, _EMB), jnp.float32),
        ),
    )(rhb, rhw, rhs, rtb, rtw, rts, eb, ebp)


def _sc_body(pos, neg, box_t, t1, t2,
             pos_ent, pos_box, neg_ent, neg_box,
             pih, pir, pit, nih, nir, nit,
             bb0, bb1, bb2, ea0, ea1, ea2, eb0, eb1, eb2,
             bsem0, bsem1, bsem2, esem0, esem1, esem2,
             wsem0, wsem1, wsem2):
    bb = (bb0, bb1, bb2)
    ea = (ea0, ea1, ea2)
    ebuf = (eb0, eb1, eb2)
    bsem = (bsem0, bsem1, bsem2)
    esem = (esem0, esem1, esem2)
    wsem = (wsem0, wsem1, wsem2)

    wid = lax.axis_index("s") * _NC + lax.axis_index("c")
    n0 = wid * _NEG_ROWS_PER_W

    p0 = wid * _POS_PER_W
    pltpu.sync_copy(pos.at[pl.ds(0 * _BATCH + p0, _POS_PER_W)], pih)
    pltpu.sync_copy(pos.at[pl.ds(1 * _BATCH + p0, _POS_PER_W)], pir)
    pltpu.sync_copy(pos.at[pl.ds(2 * _BATCH + p0, _POS_PER_W)], pit)
    for rr in range(_NEG_ROWS_PER_W):
        dst = pl.ds(rr * _BATCH, _BATCH)
        src0 = (n0 + rr) * 3 * _BATCH
        pltpu.sync_copy(neg.at[pl.ds(src0 + 0 * _BATCH, _BATCH)], nih.at[dst])
        pltpu.sync_copy(neg.at[pl.ds(src0 + 1 * _BATCH, _BATCH)], nir.at[dst])
        pltpu.sync_copy(neg.at[pl.ds(src0 + 2 * _BATCH, _BATCH)], nit.at[dst])

    def issue(slot, ih, ir, itr, off, first):
        if not first:
            pltpu.make_async_copy(bb[slot], neg_box.at[0, pl.ds(0, _C)],
                                  wsem[slot]).wait()
            pltpu.make_async_copy(ea[slot], neg_ent.at[0, pl.ds(0, _C)],
                                  wsem[slot]).wait()
        pltpu.async_copy(box_t.at[ir.at[pl.ds(off, _C)]], bb[slot],
                         bsem[slot])
        pltpu.async_copy(t1.at[ih.at[pl.ds(off, _C)]], ea[slot], esem[slot])
        pltpu.async_copy(t2.at[itr.at[pl.ds(off, _C)]], ebuf[slot],
                         esem[slot])

    def finish(slot, ent_out, box_out, n, col):
        pltpu.make_async_copy(box_t.at[pl.ds(0, _C)], bb[slot],
                              bsem[slot]).wait()
        pltpu.async_copy(bb[slot], box_out.at[n, pl.ds(col, _C)], wsem[slot])
        pltpu.make_async_copy(t1.at[pl.ds(0, _C)], ea[slot], esem[slot]).wait()
        pltpu.make_async_copy(t2.at[pl.ds(0, _C)], ebuf[slot],
                              esem[slot]).wait()

        def add_row(i, carry):
            for j in range(2):
                for k in range(_EMB // 16):
                    plsc.addupdate(ea[slot].at[i, j, pl.ds(16 * k, 16)],
                                   ebuf[slot][i, j, pl.ds(16 * k, 16)])
            return carry

        lax.fori_loop(0, _C, add_row, 0)
        pltpu.async_copy(ea[slot], ent_out.at[n, pl.ds(col, _C)], wsem[slot])

    _NPOS = _POS_PER_W // _C

    def neg_coords(j):
        flat = j * _C
        n_off = flat // _BATCH
        return n0 + n_off, flat - n_off * _BATCH

    def issue_g(g, first=False):
        if g < _NPOS:
            issue(g % 3, pih, pir, pit, g * _C, first)
        else:
            issue(g % 3, nih, nir, nit, (g - _NPOS) * _C, first)

    def finish_g(g):
        if g < _NPOS:
            finish(g % 3, pos_ent, pos_box, 0, p0 + g * _C)
        else:
            n, col = neg_coords(g - _NPOS)
            finish(g % 3, neg_ent, neg_box, n, col)

    issue_g(0, True)
    issue_g(1, True)
    for g in range(6):
        finish_g(g)
        issue_g(g + 2, first=(g + 2 == 2))

    def step(gg, carry):
        for b in range(3):
            g = 6 + 3 * gg + b
            j = g - _NPOS
            n, col = neg_coords(j)
            finish(b, neg_ent, neg_box, n, col)
            issue((b + 2) % 3, nih, nir, nit, (j + 2) * _C, False)
        return carry

    total = _NPOS + _NEG_CHUNKS
    k = (total - 8) // 3
    lax.fori_loop(0, k, step, 0)
    for g in range(6 + 3 * k, total):
        finish_g(g)
        if 8 + 3 * k <= g + 2 < total:
            issue_g(g + 2)
    for slot in (0, 1, 2):
        pltpu.make_async_copy(bb[slot], neg_box.at[0, pl.ds(0, _C)],
                              wsem[slot]).wait()
        pltpu.make_async_copy(ea[slot], neg_ent.at[0, pl.ds(0, _C)],
                              wsem[slot]).wait()


@functools.cache
def _sc_run():
  return functools.partial(
    pl.kernel,
    mesh=plsc.VectorSubcoreMesh(core_axis_name="c", subcore_axis_name="s"),
    out_type=[
        jax.ShapeDtypeStruct((1, _BATCH, 2, _EMB), jnp.float32),
        jax.ShapeDtypeStruct((1, _BATCH, 2, 2, _EMB), jnp.float32),
        jax.ShapeDtypeStruct((_NB_NEG, _BATCH, 2, _EMB), jnp.float32),
        jax.ShapeDtypeStruct((_NB_NEG, _BATCH, 2, 2, _EMB), jnp.float32),
    ],
    scratch_types=(
        [pltpu.VMEM((_POS_PER_W,), jnp.int32)] * 3
        + [pltpu.VMEM((_NEG_ROWS_PER_W * _BATCH,), jnp.int32)] * 3
        + [pltpu.VMEM((_C, 2, 2, _EMB), jnp.float32)] * 3
        + [pltpu.VMEM((_C, 2, _EMB), jnp.float32)] * 6
        + [pltpu.SemaphoreType.DMA] * 9
    ),
  )(_sc_body)


def kernel(positives, negatives, r_head_base_points, r_head_widths,
           r_head_size_scales, r_tail_base_points, r_tail_widths,
           r_tail_size_scales, entity_bases, entity_bumps):
    box_t, t1, t2 = _make_tables(
        r_head_base_points, r_head_widths, r_head_size_scales,
        r_tail_base_points, r_tail_widths, r_tail_size_scales,
        entity_bases, entity_bumps)
    pos_ent, pos_box, neg_ent, neg_box = _sc_run()(
        positives.reshape(-1), negatives.reshape(-1), box_t, t1, t2)
    return (pos_ent, pos_box, neg_ent, neg_box)
